# baseline (speedup 1.0000x reference)
import functools
import math

import jax
import jax.numpy as jnp
import numpy as np
from jax import lax
from jax.experimental import pallas as pl
from jax.experimental.pallas import tpu as pltpu
from jax.experimental.pallas import tpu_sc as plsc

INPUT_DIM = 768
HIDDEN_DIM = 8192
N_SAMPLE = 32
BATCH = 128

_TINY = np.float32(np.finfo(np.float32).tiny)
_COEFF = np.float32(1.0 / math.sqrt(2.0 * math.pi))

_SC_CORES = 2
_SC_SUBCORES = 16
_SC_WORKERS = _SC_CORES * _SC_SUBCORES


_T_BLK = 512


def _encode_body(x_ref, w_ref, h_ref, z_ref, wt_ref):
    i = pl.program_id(0)

    @pl.when(i == 0)
    def _():
        h = jnp.dot(x_ref[...], w_ref[...],
                    preferred_element_type=jnp.float32)
        h_ref[...] = h
        m = jnp.max(h, axis=-1, keepdims=True)
        e = jnp.exp(h - m)
        z_ref[...] = e / jnp.sum(e, axis=-1, keepdims=True)

    wt_ref[...] = w_ref[:, pl.ds(i * _T_BLK, _T_BLK)].T


def _encode(x, W):
    return pl.pallas_call(
        _encode_body,
        grid=(HIDDEN_DIM // _T_BLK,),
        in_specs=[
            pl.BlockSpec((BATCH, INPUT_DIM), lambda i: (0, 0)),
            pl.BlockSpec((INPUT_DIM, HIDDEN_DIM), lambda i: (0, 0)),
        ],
        out_specs=[
            pl.BlockSpec((BATCH, HIDDEN_DIM), lambda i: (0, 0)),
            pl.BlockSpec((BATCH, HIDDEN_DIM), lambda i: (0, 0)),
            pl.BlockSpec((_T_BLK, INPUT_DIM), lambda i: (i, 0)),
        ],
        out_shape=[
            jax.ShapeDtypeStruct((BATCH, HIDDEN_DIM), jnp.float32),
            jax.ShapeDtypeStruct((BATCH, HIDDEN_DIM), jnp.float32),
            jax.ShapeDtypeStruct((HIDDEN_DIM, INPUT_DIM), jnp.float32),
        ],
        interpret=False,
    )(x, W)


def _threefry_bits(m):
    ks = (
        jnp.uint32(0),
        jnp.uint32(1234),
        jnp.uint32(0) ^ jnp.uint32(1234) ^ jnp.uint32(0x1BD11BDA),
    )
    rot = ((13, 15, 26, 6), (17, 29, 16, 24))
    x0 = jnp.full_like(m, ks[0])
    x1 = m + ks[1]
    for r5 in range(5):
        for r in rot[r5 % 2]:
            x0 = x0 + x1
            x1 = (x1 << jnp.uint32(r)) | (x1 >> jnp.uint32(32 - r))
            x1 = x0 ^ x1
        x0 = x0 + ks[(r5 + 1) % 3]
        x1 = x1 + ks[(r5 + 2) % 3] + jnp.uint32(r5 + 1)
    return x0 ^ x1


def _gumbel_from_bits(bits):
    fb = (bits >> jnp.uint32(9)) | jnp.uint32(0x3F800000)
    f = lax.bitcast_convert_type(fb, jnp.float32) - jnp.float32(1.0)
    u = jnp.maximum(jnp.float32(_TINY), f + jnp.float32(_TINY))
    return -jnp.log(-jnp.log(u))


_SAMPLE_CHUNK = 2048

_K_TC = 22
_S_SC = N_SAMPLE - _K_TC
_S_A = _S_SC // 2
_S_B = _S_SC - _S_A


def _sample_body(z_ref, idx_ref, oh_ref):
    s = pl.program_id(0)
    su = s.astype(jnp.uint32)
    row = lax.broadcasted_iota(jnp.uint32, (BATCH, _SAMPLE_CHUNK), 0)
    col = lax.broadcasted_iota(jnp.uint32, (BATCH, _SAMPLE_CHUNK), 1)
    coli = lax.broadcasted_iota(jnp.int32, (BATCH, _SAMPLE_CHUNK), 1)
    base = su * jnp.uint32(BATCH * HIDDEN_DIM) + row * jnp.uint32(HIDDEN_DIM)
    best_v = jnp.full((BATCH, 1), -jnp.inf, jnp.float32)
    best_i = jnp.zeros((BATCH, 1), jnp.int32)
    for c in range(HIDDEN_DIM // _SAMPLE_CHUNK):
        off = c * _SAMPLE_CHUNK
        mctr = base + col + jnp.uint32(off)
        g = _gumbel_from_bits(_threefry_bits(mctr))
        val = g + z_ref[:, off:off + _SAMPLE_CHUNK]
        mx = jnp.max(val, axis=-1, keepdims=True)
        ic = jnp.min(
            jnp.where(val == mx, coli + jnp.int32(off), jnp.int32(HIDDEN_DIM)),
            axis=-1, keepdims=True)
        upd = mx > best_v
        best_i = jnp.where(upd, ic, best_i)
        best_v = jnp.where(upd, mx, best_v)
    idx_ref[0] = best_i
    ohcol = lax.broadcasted_iota(jnp.int32, (BATCH, HIDDEN_DIM), 1)
    oh_ref[0] = (ohcol == best_i).astype(jnp.float32)


def _sample(z):
    return pl.pallas_call(
        _sample_body,
        grid=(_K_TC,),
        in_specs=[pl.BlockSpec((BATCH, HIDDEN_DIM), lambda s: (0, 0))],
        out_specs=[
            pl.BlockSpec((1, BATCH, 1), lambda s: (s, 0, 0)),
            pl.BlockSpec((1, BATCH, HIDDEN_DIM), lambda s: (s, 0, 0)),
        ],
        out_shape=[
            jax.ShapeDtypeStruct((_K_TC, BATCH, 1), jnp.int32),
            jax.ShapeDtypeStruct((N_SAMPLE, BATCH, HIDDEN_DIM), jnp.float32),
        ],
        interpret=False,
    )(z)


_SC_L = 16
_SC_UNROLL = 16


def _sc_bits_body(s0, n_rows, bits_hbm, buf0, buf1, sem0, sem1):
    rpw = (n_rows * BATCH) // _SC_WORKERS
    wid = lax.axis_index("s") * _SC_CORES + lax.axis_index("c")
    lane = lax.iota(jnp.int32, _SC_L).astype(jnp.uint32)
    bufs = (buf0, buf1)
    sems = (sem0, sem1)

    def outer(ro, _):
        for p in range(2):
            r = ro * 2 + p
            flat_row = wid * rpw + r
            base = (jnp.uint32(s0 * BATCH)
                    + flat_row.astype(jnp.uint32)) * jnp.uint32(HIDDEN_DIM)
            buf = bufs[p]
            sem = sems[p]

            @pl.when(ro > 0)
            def _wait():
                pltpu.make_async_copy(buf, bits_hbm.at[0], sem).wait()

            @plsc.parallel_loop(0, HIDDEN_DIM, _SC_L, unroll=_SC_UNROLL)
            def _chunk(off):
                m = lane + (base + off.astype(jnp.uint32))
                buf[pl.ds(off, _SC_L)] = _threefry_bits(m)

            pltpu.make_async_copy(buf, bits_hbm.at[flat_row], sem).start()
        return 0

    lax.fori_loop(0, rpw // 2, outer, 0)
    for p in range(2):
        pltpu.make_async_copy(bufs[p], bits_hbm.at[0], sems[p]).wait()


def _sc_bits(s0, n_rows):
    run = pl.kernel(
        functools.partial(_sc_bits_body, s0, n_rows),
        out_type=jax.ShapeDtypeStruct((n_rows * BATCH, HIDDEN_DIM),
                                      jnp.uint32),
        mesh=plsc.VectorSubcoreMesh(core_axis_name="c", subcore_axis_name="s"),
        scratch_types=[
            pltpu.VMEM((HIDDEN_DIM,), jnp.uint32),
            pltpu.VMEM((HIDDEN_DIM,), jnp.uint32),
            pltpu.SemaphoreType.DMA,
            pltpu.SemaphoreType.DMA,
        ],
    )
    return run()


def _score_body(bits_ref, z_ref, oh_in_ref, idx_ref, oh_ref):
    del oh_in_ref
    coli = lax.broadcasted_iota(jnp.int32, (BATCH, _SAMPLE_CHUNK), 1)
    best_v = jnp.full((BATCH, 1), -jnp.inf, jnp.float32)
    best_i = jnp.zeros((BATCH, 1), jnp.int32)
    for c in range(HIDDEN_DIM // _SAMPLE_CHUNK):
        off = c * _SAMPLE_CHUNK
        g = _gumbel_from_bits(bits_ref[:, off:off + _SAMPLE_CHUNK])
        val = g + z_ref[:, off:off + _SAMPLE_CHUNK]
        mx = jnp.max(val, axis=-1, keepdims=True)
        ic = jnp.min(
            jnp.where(val == mx, coli + jnp.int32(off), jnp.int32(HIDDEN_DIM)),
            axis=-1, keepdims=True)
        upd = mx > best_v
        best_i = jnp.where(upd, ic, best_i)
        best_v = jnp.where(upd, mx, best_v)
    idx_ref[0] = best_i
    ohcol = lax.broadcasted_iota(jnp.int32, (BATCH, HIDDEN_DIM), 1)
    oh_ref[0] = (ohcol == best_i).astype(jnp.float32)


def _score(bits, z, oh_partial, s0, n_steps):
    return pl.pallas_call(
        _score_body,
        grid=(n_steps,),
        in_specs=[
            pl.BlockSpec((BATCH, HIDDEN_DIM), lambda i: (i, 0)),
            pl.BlockSpec((BATCH, HIDDEN_DIM), lambda i: (0, 0)),
            pl.BlockSpec(memory_space=pl.ANY),
        ],
        out_specs=[
            pl.BlockSpec((1, BATCH, 1), lambda i: (i, 0, 0)),
            pl.BlockSpec((1, BATCH, HIDDEN_DIM), lambda i: (i + s0, 0, 0)),
        ],
        out_shape=[
            jax.ShapeDtypeStruct((n_steps, BATCH, 1), jnp.int32),
            jax.ShapeDtypeStruct((N_SAMPLE, BATCH, HIDDEN_DIM), jnp.float32),
        ],
        input_output_aliases={2: 1},
        interpret=False,
    )(bits, z, oh_partial)


_ROWS_PER_WORKER = (N_SAMPLE * BATCH) // _SC_WORKERS


def _gather_sc_body(wt_hbm, idx_hbm, out_hbm, idx_v, rows_v, sem):
    wid = lax.axis_index("s") * _SC_CORES + lax.axis_index("c")
    base = wid * _ROWS_PER_WORKER
    pltpu.sync_copy(idx_hbm.at[pl.ds(base, _ROWS_PER_WORKER)], idx_v)
    pltpu.async_copy(wt_hbm.at[idx_v], rows_v, sem).wait()
    pltpu.sync_copy(rows_v, out_hbm.at[pl.ds(base, _ROWS_PER_WORKER)])


def _gather_rows(Wt, idx_flat):
    run = pl.kernel(
        _gather_sc_body,
        out_type=jax.ShapeDtypeStruct((N_SAMPLE * BATCH, INPUT_DIM),
                                      jnp.float32),
        mesh=plsc.VectorSubcoreMesh(core_axis_name="c", subcore_axis_name="s"),
        scratch_types=[
            pltpu.VMEM((_ROWS_PER_WORKER,), jnp.int32),
            pltpu.VMEM((_ROWS_PER_WORKER, INPUT_DIM), jnp.float32),
            pltpu.SemaphoreType.DMA,
        ],
    )
    return run(Wt, idx_flat)


_G_BLK = 8


def _weight_body(x_ref, xd_ref, w_ref):
    x = x_ref[...]
    temps = []
    for i in range(N_SAMPLE // _G_BLK):
        d = xd_ref[i * _G_BLK:(i + 1) * _G_BLK] - x[None]
        s2 = jnp.sum(d * d, axis=2)
        temps.append(jnp.float32(_COEFF) * jnp.exp(-0.5 * s2))
    t = jnp.concatenate(temps, axis=0)
    w_ref[...] = t / jnp.mean(t, axis=0, keepdims=True)


def _weight(x, xd3):
    return pl.pallas_call(
        _weight_body,
        out_shape=jax.ShapeDtypeStruct((N_SAMPLE, BATCH), jnp.float32),
        interpret=False,
    )(x, xd3)


def kernel(x, W):
    bits_a = _sc_bits(_K_TC, _S_A)
    bits_b = _sc_bits(_K_TC + _S_A, _S_B)
    h, z, Wt = _encode(x, W)
    idx_lo, oh_partial = _sample(z)
    idx_a, oh_a = _score(bits_a, z, oh_partial, _K_TC, _S_A)
    idx_b, onehot = _score(bits_b, z, oh_a, _K_TC + _S_A, _S_B)
    idx3 = jnp.concatenate([idx_lo, idx_a, idx_b], axis=0)
    idx_flat = idx3.reshape(N_SAMPLE * BATCH)
    xd_flat = _gather_rows(Wt, idx_flat)
    x_decoded = xd_flat.reshape(N_SAMPLE, BATCH, INPUT_DIM)
    weight = _weight(x, x_decoded)
    return (h, z, onehot, x_decoded, weight)


# speedup vs baseline: 1.0018x; 1.0018x over previous
import functools
import math

import jax
import jax.numpy as jnp
import numpy as np
from jax import lax
from jax.experimental import pallas as pl
from jax.experimental.pallas import tpu as pltpu
from jax.experimental.pallas import tpu_sc as plsc

INPUT_DIM = 768
HIDDEN_DIM = 8192
N_SAMPLE = 32
BATCH = 128

_TINY = np.float32(np.finfo(np.float32).tiny)
_COEFF = np.float32(1.0 / math.sqrt(2.0 * math.pi))

_SC_CORES = 2
_SC_SUBCORES = 16
_SC_WORKERS = _SC_CORES * _SC_SUBCORES


_T_BLK = 512


def _encode_body(x_ref, w_ref, h_ref, z_ref, wt_ref):
    i = pl.program_id(0)

    @pl.when(i == 0)
    def _():
        h = jnp.dot(x_ref[...], w_ref[...],
                    preferred_element_type=jnp.float32)
        h_ref[...] = h
        m = jnp.max(h, axis=-1, keepdims=True)
        e = jnp.exp(h - m)
        z_ref[...] = e / jnp.sum(e, axis=-1, keepdims=True)

    wt_ref[...] = w_ref[:, pl.ds(i * _T_BLK, _T_BLK)].T


def _encode(x, W):
    return pl.pallas_call(
        _encode_body,
        grid=(HIDDEN_DIM // _T_BLK,),
        in_specs=[
            pl.BlockSpec((BATCH, INPUT_DIM), lambda i: (0, 0)),
            pl.BlockSpec((INPUT_DIM, HIDDEN_DIM), lambda i: (0, 0)),
        ],
        out_specs=[
            pl.BlockSpec((BATCH, HIDDEN_DIM), lambda i: (0, 0)),
            pl.BlockSpec((BATCH, HIDDEN_DIM), lambda i: (0, 0)),
            pl.BlockSpec((_T_BLK, INPUT_DIM), lambda i: (i, 0)),
        ],
        out_shape=[
            jax.ShapeDtypeStruct((BATCH, HIDDEN_DIM), jnp.float32),
            jax.ShapeDtypeStruct((BATCH, HIDDEN_DIM), jnp.float32),
            jax.ShapeDtypeStruct((HIDDEN_DIM, INPUT_DIM), jnp.float32),
        ],
        interpret=False,
    )(x, W)


def _threefry_bits(m):
    ks = (
        jnp.uint32(0),
        jnp.uint32(1234),
        jnp.uint32(0) ^ jnp.uint32(1234) ^ jnp.uint32(0x1BD11BDA),
    )
    rot = ((13, 15, 26, 6), (17, 29, 16, 24))
    x0 = jnp.full_like(m, ks[0])
    x1 = m + ks[1]
    for r5 in range(5):
        for r in rot[r5 % 2]:
            x0 = x0 + x1
            x1 = (x1 << jnp.uint32(r)) | (x1 >> jnp.uint32(32 - r))
            x1 = x0 ^ x1
        x0 = x0 + ks[(r5 + 1) % 3]
        x1 = x1 + ks[(r5 + 2) % 3] + jnp.uint32(r5 + 1)
    return x0 ^ x1


def _gumbel_from_bits(bits):
    fb = (bits >> jnp.uint32(9)) | jnp.uint32(0x3F800000)
    f = lax.bitcast_convert_type(fb, jnp.float32) - jnp.float32(1.0)
    u = jnp.maximum(jnp.float32(_TINY), f + jnp.float32(_TINY))
    return -jnp.log(-jnp.log(u))


_SAMPLE_CHUNK = 2048

_K_TC = 22
_S_SC = N_SAMPLE - _K_TC
_S_A = 6
_S_B = _S_SC - _S_A


def _sample_body(z_ref, idx_ref, oh_ref):
    s = pl.program_id(0)
    su = s.astype(jnp.uint32)
    row = lax.broadcasted_iota(jnp.uint32, (BATCH, _SAMPLE_CHUNK), 0)
    col = lax.broadcasted_iota(jnp.uint32, (BATCH, _SAMPLE_CHUNK), 1)
    coli = lax.broadcasted_iota(jnp.int32, (BATCH, _SAMPLE_CHUNK), 1)
    base = su * jnp.uint32(BATCH * HIDDEN_DIM) + row * jnp.uint32(HIDDEN_DIM)
    best_v = jnp.full((BATCH, 1), -jnp.inf, jnp.float32)
    best_i = jnp.zeros((BATCH, 1), jnp.int32)
    for c in range(HIDDEN_DIM // _SAMPLE_CHUNK):
        off = c * _SAMPLE_CHUNK
        mctr = base + col + jnp.uint32(off)
        g = _gumbel_from_bits(_threefry_bits(mctr))
        val = g + z_ref[:, off:off + _SAMPLE_CHUNK]
        mx = jnp.max(val, axis=-1, keepdims=True)
        ic = jnp.min(
            jnp.where(val == mx, coli + jnp.int32(off), jnp.int32(HIDDEN_DIM)),
            axis=-1, keepdims=True)
        upd = mx > best_v
        best_i = jnp.where(upd, ic, best_i)
        best_v = jnp.where(upd, mx, best_v)
    idx_ref[0] = best_i
    ohcol = lax.broadcasted_iota(jnp.int32, (BATCH, HIDDEN_DIM), 1)
    oh_ref[0] = (ohcol == best_i).astype(jnp.float32)


def _sample(z):
    return pl.pallas_call(
        _sample_body,
        grid=(_K_TC,),
        in_specs=[pl.BlockSpec((BATCH, HIDDEN_DIM), lambda s: (0, 0))],
        out_specs=[
            pl.BlockSpec((1, BATCH, 1), lambda s: (s, 0, 0)),
            pl.BlockSpec((1, BATCH, HIDDEN_DIM), lambda s: (s, 0, 0)),
        ],
        out_shape=[
            jax.ShapeDtypeStruct((_K_TC, BATCH, 1), jnp.int32),
            jax.ShapeDtypeStruct((N_SAMPLE, BATCH, HIDDEN_DIM), jnp.float32),
        ],
        interpret=False,
    )(z)


_SC_L = 16
_SC_UNROLL = 16


def _sc_bits_body(s0, n_rows, bits_hbm, buf0, buf1, sem0, sem1):
    rpw = (n_rows * BATCH) // _SC_WORKERS
    wid = lax.axis_index("s") * _SC_CORES + lax.axis_index("c")
    lane = lax.iota(jnp.int32, _SC_L).astype(jnp.uint32)
    bufs = (buf0, buf1)
    sems = (sem0, sem1)

    def outer(ro, _):
        for p in range(2):
            r = ro * 2 + p
            flat_row = wid * rpw + r
            base = (jnp.uint32(s0 * BATCH)
                    + flat_row.astype(jnp.uint32)) * jnp.uint32(HIDDEN_DIM)
            buf = bufs[p]
            sem = sems[p]

            @pl.when(ro > 0)
            def _wait():
                pltpu.make_async_copy(buf, bits_hbm.at[0], sem).wait()

            @plsc.parallel_loop(0, HIDDEN_DIM, _SC_L, unroll=_SC_UNROLL)
            def _chunk(off):
                m = lane + (base + off.astype(jnp.uint32))
                buf[pl.ds(off, _SC_L)] = _threefry_bits(m)

            pltpu.make_async_copy(buf, bits_hbm.at[flat_row], sem).start()
        return 0

    lax.fori_loop(0, rpw // 2, outer, 0)
    for p in range(2):
        pltpu.make_async_copy(bufs[p], bits_hbm.at[0], sems[p]).wait()


def _sc_bits(s0, n_rows):
    run = pl.kernel(
        functools.partial(_sc_bits_body, s0, n_rows),
        out_type=jax.ShapeDtypeStruct((n_rows * BATCH, HIDDEN_DIM),
                                      jnp.uint32),
        mesh=plsc.VectorSubcoreMesh(core_axis_name="c", subcore_axis_name="s"),
        scratch_types=[
            pltpu.VMEM((HIDDEN_DIM,), jnp.uint32),
            pltpu.VMEM((HIDDEN_DIM,), jnp.uint32),
            pltpu.SemaphoreType.DMA,
            pltpu.SemaphoreType.DMA,
        ],
    )
    return run()


def _score_body(bits_ref, z_ref, oh_in_ref, idx_ref, oh_ref):
    del oh_in_ref
    coli = lax.broadcasted_iota(jnp.int32, (BATCH, _SAMPLE_CHUNK), 1)
    best_v = jnp.full((BATCH, 1), -jnp.inf, jnp.float32)
    best_i = jnp.zeros((BATCH, 1), jnp.int32)
    for c in range(HIDDEN_DIM // _SAMPLE_CHUNK):
        off = c * _SAMPLE_CHUNK
        g = _gumbel_from_bits(bits_ref[:, off:off + _SAMPLE_CHUNK])
        val = g + z_ref[:, off:off + _SAMPLE_CHUNK]
        mx = jnp.max(val, axis=-1, keepdims=True)
        ic = jnp.min(
            jnp.where(val == mx, coli + jnp.int32(off), jnp.int32(HIDDEN_DIM)),
            axis=-1, keepdims=True)
        upd = mx > best_v
        best_i = jnp.where(upd, ic, best_i)
        best_v = jnp.where(upd, mx, best_v)
    idx_ref[0] = best_i
    ohcol = lax.broadcasted_iota(jnp.int32, (BATCH, HIDDEN_DIM), 1)
    oh_ref[0] = (ohcol == best_i).astype(jnp.float32)


def _score(bits, z, oh_partial, s0, n_steps):
    return pl.pallas_call(
        _score_body,
        grid=(n_steps,),
        in_specs=[
            pl.BlockSpec((BATCH, HIDDEN_DIM), lambda i: (i, 0)),
            pl.BlockSpec((BATCH, HIDDEN_DIM), lambda i: (0, 0)),
            pl.BlockSpec(memory_space=pl.ANY),
        ],
        out_specs=[
            pl.BlockSpec((1, BATCH, 1), lambda i: (i, 0, 0)),
            pl.BlockSpec((1, BATCH, HIDDEN_DIM), lambda i: (i + s0, 0, 0)),
        ],
        out_shape=[
            jax.ShapeDtypeStruct((n_steps, BATCH, 1), jnp.int32),
            jax.ShapeDtypeStruct((N_SAMPLE, BATCH, HIDDEN_DIM), jnp.float32),
        ],
        input_output_aliases={2: 1},
        interpret=False,
    )(bits, z, oh_partial)


_ROWS_PER_WORKER = (N_SAMPLE * BATCH) // _SC_WORKERS


def _gather_sc_body(wt_hbm, idx_hbm, out_hbm, idx_v, rows_v, sem):
    wid = lax.axis_index("s") * _SC_CORES + lax.axis_index("c")
    base = wid * _ROWS_PER_WORKER
    pltpu.sync_copy(idx_hbm.at[pl.ds(base, _ROWS_PER_WORKER)], idx_v)
    pltpu.async_copy(wt_hbm.at[idx_v], rows_v, sem).wait()
    pltpu.sync_copy(rows_v, out_hbm.at[pl.ds(base, _ROWS_PER_WORKER)])


def _gather_rows(Wt, idx_flat):
    run = pl.kernel(
        _gather_sc_body,
        out_type=jax.ShapeDtypeStruct((N_SAMPLE * BATCH, INPUT_DIM),
                                      jnp.float32),
        mesh=plsc.VectorSubcoreMesh(core_axis_name="c", subcore_axis_name="s"),
        scratch_types=[
            pltpu.VMEM((_ROWS_PER_WORKER,), jnp.int32),
            pltpu.VMEM((_ROWS_PER_WORKER, INPUT_DIM), jnp.float32),
            pltpu.SemaphoreType.DMA,
        ],
    )
    return run(Wt, idx_flat)


_G_BLK = 8


def _weight_body(x_ref, xd_ref, w_ref):
    x = x_ref[...]
    temps = []
    for i in range(N_SAMPLE // _G_BLK):
        d = xd_ref[i * _G_BLK:(i + 1) * _G_BLK] - x[None]
        s2 = jnp.sum(d * d, axis=2)
        temps.append(jnp.float32(_COEFF) * jnp.exp(-0.5 * s2))
    t = jnp.concatenate(temps, axis=0)
    w_ref[...] = t / jnp.mean(t, axis=0, keepdims=True)


def _weight(x, xd3):
    return pl.pallas_call(
        _weight_body,
        out_shape=jax.ShapeDtypeStruct((N_SAMPLE, BATCH), jnp.float32),
        interpret=False,
    )(x, xd3)


def kernel(x, W):
    bits_a = _sc_bits(_K_TC, _S_A)
    bits_b = _sc_bits(_K_TC + _S_A, _S_B)
    h, z, Wt = _encode(x, W)
    idx_lo, oh_partial = _sample(z)
    idx_a, oh_a = _score(bits_a, z, oh_partial, _K_TC, _S_A)
    idx_b, onehot = _score(bits_b, z, oh_a, _K_TC + _S_A, _S_B)
    idx3 = jnp.concatenate([idx_lo, idx_a, idx_b], axis=0)
    idx_flat = idx3.reshape(N_SAMPLE * BATCH)
    xd_flat = _gather_rows(Wt, idx_flat)
    x_decoded = xd_flat.reshape(N_SAMPLE, BATCH, INPUT_DIM)
    weight = _weight(x, x_decoded)
    return (h, z, onehot, x_decoded, weight)
